# baseline (speedup 1.0000x reference)
import functools
import numpy as np
import jax
import jax.numpy as jnp
from jax import lax
from jax.experimental import pallas as pl
from jax.experimental.pallas import tpu as pltpu
from jax.experimental.pallas import tpu_sc as plsc

_NUM_BUCKETS = 32
_MAX_DISTANCE = 128

_BQ = 1024
_G = 128

_NC = 2
_NS = 16
_L = 16


def _sc_bias_body(table_ref, out_ref, table_v, row_v, *, q_len, width,
                  num_heads, per_w):
    wid = lax.axis_index("s") * _NC + lax.axis_index("c")
    base = wid * per_w
    pltpu.sync_copy(table_ref, table_v)

    def step(i, _):
        f = base + i * _L + lax.iota(jnp.int32, _L)
        j = lax.rem(f, width)
        n = (q_len - 1) - j
        half = _NUM_BUCKETS // 2
        ret = jnp.where(n < 0, half, 0)
        na = jnp.abs(n)
        max_exact = half // 2
        nsq = na * na
        val_large = jnp.full((_L,), max_exact - 6, dtype=jnp.int32)
        one = jnp.ones((_L,), dtype=jnp.int32)
        zero = jnp.zeros((_L,), dtype=jnp.int32)
        for m in range(1, 14):
            val_large = val_large + jnp.where(nsq >= (1 << m), one, zero)
        bucket = ret + jnp.where(na < max_exact, na, val_large)
        h = lax.div(f, width)
        idx = bucket * num_heads + h
        row_v[pl.ds(i * _L, _L)] = plsc.load_gather(table_v, [idx])
        return 0

    lax.fori_loop(0, per_w // _L, step, 0)
    pltpu.sync_copy(row_v, out_ref.at[pl.ds(base, per_w)])


def _tc_add_body(bias_ref, in_ref, o_ref, s_ref, *, q_len, k_len, width):
    a = pl.program_id(1)

    @pl.when(a == 0)
    def _expand_bias_diagonals():
        s_ref[...] = pltpu.roll(
            jnp.broadcast_to(bias_ref[0], (_G, width)),
            width - (_G - 1), 1, stride=1, stride_axis=0,
        )

    for g in range(_BQ // _G):
        m0 = (q_len - _G) - _G * ((_BQ // _G) * a + g)
        o_ref[0, 0, pl.ds(_G * g, _G), :] = (
            in_ref[0, 0, pl.ds(_G * g, _G), :] + s_ref[:, pl.ds(m0, k_len)]
        )


def kernel(inputs, rel_embedding):
    b, num_heads, q_len, k_len = inputs.shape
    width = ((q_len + k_len - 1 + 127) // 128) * 128
    per_w = (num_heads * width) // (_NC * _NS)

    sc_bias = pl.kernel(
        functools.partial(_sc_bias_body, q_len=q_len, width=width,
                          num_heads=num_heads, per_w=per_w),
        out_type=jax.ShapeDtypeStruct((num_heads * width,), jnp.float32),
        mesh=plsc.VectorSubcoreMesh(core_axis_name="c", subcore_axis_name="s",
                                    num_cores=_NC, num_subcores=_NS),
        scratch_types=[
            pltpu.VMEM((_NUM_BUCKETS * num_heads,), jnp.float32),
            pltpu.VMEM((per_w,), jnp.float32),
        ],
        compiler_params=pltpu.CompilerParams(needs_layout_passes=False),
    )
    bias_rows = sc_bias(rel_embedding.reshape(-1))
    bias_rows = bias_rows.reshape(num_heads, 1, width)

    out = pl.pallas_call(
        functools.partial(_tc_add_body, q_len=q_len, k_len=k_len,
                          width=width),
        grid=(num_heads, q_len // _BQ),
        in_specs=[
            pl.BlockSpec((1, 1, width), lambda h, a: (h, 0, 0)),
            pl.BlockSpec((1, 1, _BQ, k_len), lambda h, a: (0, h, a, 0)),
        ],
        out_specs=pl.BlockSpec((1, 1, _BQ, k_len), lambda h, a: (0, h, a, 0)),
        out_shape=jax.ShapeDtypeStruct(inputs.shape, inputs.dtype),
        scratch_shapes=[pltpu.VMEM((_G, width), jnp.float32)],
        compiler_params=pltpu.CompilerParams(
            dimension_semantics=("parallel", "arbitrary"),
        ),
    )(bias_rows, inputs)
    return out


# speedup vs baseline: 1.0023x; 1.0023x over previous
import functools
import jax
import jax.numpy as jnp
from jax import lax
from jax.experimental import pallas as pl
from jax.experimental.pallas import tpu as pltpu
from jax.experimental.pallas import tpu_sc as plsc

_NUM_BUCKETS = 32
_MAX_DISTANCE = 128

_BQ = 1024
_G = 128

_NC = 2
_NS = 16
_L = 16


def _sc_bias_body(table_ref, out_ref, table_v, row_v, *, q_len, width,
                  num_heads, per_w):
    wid = lax.axis_index("s") * _NC + lax.axis_index("c")
    base = wid * per_w
    pltpu.sync_copy(table_ref, table_v)

    def step(i, _):
        f = base + i * _L + lax.iota(jnp.int32, _L)
        j = lax.rem(f, width)
        n = (q_len - 1) - j
        half = _NUM_BUCKETS // 2
        ret = jnp.where(n < 0, half, 0)
        na = jnp.abs(n)
        max_exact = half // 2
        nsq = na * na
        val_large = jnp.full((_L,), max_exact - 6, dtype=jnp.int32)
        one = jnp.ones((_L,), dtype=jnp.int32)
        zero = jnp.zeros((_L,), dtype=jnp.int32)
        for m in range(1, 14):
            val_large = val_large + jnp.where(nsq >= (1 << m), one, zero)
        bucket = ret + jnp.where(na < max_exact, na, val_large)
        h = lax.div(f, width)
        idx = bucket * num_heads + h
        row_v[pl.ds(i * _L, _L)] = plsc.load_gather(table_v, [idx])
        return 0

    lax.fori_loop(0, per_w // _L, step, 0)
    pltpu.sync_copy(row_v, out_ref.at[pl.ds(base, per_w)])


def _tc_add_body(bias_ref, in_ref, o_ref, s_ref, *, q_len, k_len, width):
    a = pl.program_id(1)

    @pl.when(a == 0)
    def _expand_bias_diagonals():
        s_ref[...] = pltpu.roll(
            jnp.broadcast_to(bias_ref[0], (_G, width)),
            width - (_G - 1), 1, stride=1, stride_axis=0,
        )

    for g in range(_BQ // _G):
        m0 = (q_len - _G) - _G * ((_BQ // _G) * a + g)
        o_ref[0, 0, pl.ds(_G * g, _G), :] = (
            in_ref[0, 0, pl.ds(_G * g, _G), :] + s_ref[:, pl.ds(m0, k_len)]
        )


def kernel(inputs, rel_embedding):
    b, num_heads, q_len, k_len = inputs.shape
    width = ((q_len + k_len - 1 + 127) // 128) * 128
    per_w = (num_heads * width) // (_NC * _NS)

    sc_bias = pl.kernel(
        functools.partial(_sc_bias_body, q_len=q_len, width=width,
                          num_heads=num_heads, per_w=per_w),
        out_type=jax.ShapeDtypeStruct((num_heads * width,), jnp.float32),
        mesh=plsc.VectorSubcoreMesh(core_axis_name="c", subcore_axis_name="s",
                                    num_cores=_NC, num_subcores=_NS),
        scratch_types=[
            pltpu.VMEM((_NUM_BUCKETS * num_heads,), jnp.float32),
            pltpu.VMEM((per_w,), jnp.float32),
        ],
        compiler_params=pltpu.CompilerParams(needs_layout_passes=False),
    )
    bias_rows = sc_bias(rel_embedding.reshape(-1))
    bias_rows = bias_rows.reshape(num_heads, 1, width)

    out = pl.pallas_call(
        functools.partial(_tc_add_body, q_len=q_len, k_len=k_len,
                          width=width),
        grid=(num_heads, q_len // _BQ),
        in_specs=[
            pl.BlockSpec((1, 1, width), lambda h, a: (h, 0, 0)),
            pl.BlockSpec((1, 1, _BQ, k_len), lambda h, a: (0, h, a, 0)),
        ],
        out_specs=pl.BlockSpec((1, 1, _BQ, k_len), lambda h, a: (0, h, a, 0)),
        out_shape=jax.ShapeDtypeStruct(inputs.shape, inputs.dtype),
        scratch_shapes=[pltpu.VMEM((_G, width), jnp.float32)],
        compiler_params=pltpu.CompilerParams(
            dimension_semantics=("parallel", "arbitrary"),
        ),
    )(bias_rows, inputs)
    return out
